# Initial kernel scaffold; baseline (speedup 1.0000x reference)
import jax
import jax.numpy as jnp
from jax.experimental import pallas as pl


def kernel(id_batch, neigh_cell, neigh_drug, neigh_gene, cell_features, drug_features, gene_features, W_cell, b_cell, W_drug, b_drug, W_gene, b_gene, W_l1, b_l1, W_l2, b_l2):
    raise NotImplementedError("write your pallas kernel here")


# speedup vs baseline: 1.2212x; 1.2212x over previous
import functools

import jax
import jax.numpy as jnp
from jax import lax
from jax.experimental import pallas as pl
from jax.experimental.pallas import tpu as pltpu
from jax.experimental.pallas import tpu_sc as plsc

MAX_NEIGHBORS = 10
PAD_VALUE = -1
EMBED_D = 128

_NC = 2
_NS = 16
_NW = _NC * _NS
_EPR = 4


def _sc_gather_sums(idx_c, w_c, idx_d, w_d, idx_g, w_g, ids,
                    cell_features, drug_features, gene_features):
  B = ids.shape[0]
  Dc = cell_features.shape[1]
  Dd = drug_features.shape[1]
  Dg = gene_features.shape[1]
  assert B % _NW == 0
  epw = B // _NW
  rounds = epw // _EPR
  ipw = epw * MAX_NEIGHBORS

  mesh = plsc.VectorSubcoreMesh(core_axis_name="c", subcore_axis_name="s")

  @functools.partial(
      pl.kernel,
      out_type=[
          jax.ShapeDtypeStruct((B, Dc), jnp.float32),
          jax.ShapeDtypeStruct((B, Dd), jnp.float32),
          jax.ShapeDtypeStruct((B, Dg), jnp.float32),
          jax.ShapeDtypeStruct((B, Dd), jnp.float32),
      ],
      mesh=mesh,
      scratch_types=[
          pltpu.VMEM((ipw,), jnp.int32),
          pltpu.VMEM((epw * 16,), jnp.float32),
          pltpu.VMEM((_EPR * MAX_NEIGHBORS, Dd), jnp.float32),
          pltpu.VMEM((_EPR * MAX_NEIGHBORS, Dc), jnp.float32),
          pltpu.VMEM((_EPR * MAX_NEIGHBORS, Dg), jnp.float32),
          pltpu.VMEM((_EPR, Dd), jnp.float32),
          pltpu.VMEM((_EPR, Dc), jnp.float32),
          pltpu.VMEM((_EPR, Dg), jnp.float32),
          pltpu.SemaphoreType.DMA,
      ],
  )
  def k(idx_c_h, w_c_h, idx_d_h, w_d_h, idx_g_h, w_g_h, ids_h,
        cell_h, drug_h, gene_h,
        sums_c_h, sums_d_h, sums_g_h, self_h,
        idx_v, w_v, rows_d, rows_c, rows_g, acc_d, acc_c, acc_g, sem):
    wid = lax.axis_index("s") * _NC + lax.axis_index("c")
    base_e = wid * epw
    base_i = wid * ipw

    def agg_type(idx_h, w_h, tbl_h, out_h, rows_v, acc_v, D):
      nch = D // 16
      pltpu.sync_copy(idx_h.at[pl.ds(base_i, ipw)], idx_v)
      pltpu.sync_copy(w_h.at[pl.ds(base_e * 16, epw * 16)], w_v)
      for r in range(rounds):
        i0 = r * _EPR * MAX_NEIGHBORS
        pltpu.async_copy(
            tbl_h.at[idx_v.at[pl.ds(i0, _EPR * MAX_NEIGHBORS)]],
            rows_v, sem).wait()
        wvecs = [w_v[pl.ds((r * _EPR + e) * 16, 16)] for e in range(_EPR)]

        def chunk(c, carry):
          off = pl.multiple_of(c * 16, 16)
          for e in range(_EPR):
            acc = jnp.zeros((16,), jnp.float32)
            for j in range(MAX_NEIGHBORS):
              acc = acc + rows_v[e * MAX_NEIGHBORS + j, pl.ds(off, 16)] * wvecs[e][j]
            acc_v[e, pl.ds(off, 16)] = acc
          return carry

        lax.fori_loop(0, nch, chunk, 0)
        pltpu.sync_copy(acc_v, out_h.at[pl.ds(base_e + r * _EPR, _EPR)])

    agg_type(idx_c_h, w_c_h, cell_h, sums_c_h, rows_c, acc_c, Dc)
    agg_type(idx_g_h, w_g_h, gene_h, sums_g_h, rows_g, acc_g, Dg)
    agg_type(idx_d_h, w_d_h, drug_h, sums_d_h, rows_d, acc_d, Dd)

    pltpu.sync_copy(ids_h.at[pl.ds(base_e, epw)], idx_v.at[pl.ds(0, epw)])
    chunk_rows = _EPR * MAX_NEIGHBORS
    done = 0
    while done < epw:
      n = min(chunk_rows, epw - done)
      pltpu.async_copy(
          drug_h.at[idx_v.at[pl.ds(done, n)]],
          rows_d.at[pl.ds(0, n)], sem).wait()
      pltpu.sync_copy(rows_d.at[pl.ds(0, n)],
                      self_h.at[pl.ds(base_e + done, n)])
      done += n

  return k(idx_c, w_c, idx_d, w_d, idx_g, w_g, ids,
           cell_features, drug_features, gene_features)


def _tc_mlp(self_rows, sums_c, sums_d, sums_g, cnt_c, cnt_d, cnt_g,
            W_cell, b_cell, W_drug, b_drug, W_gene, b_gene,
            W_l1, b_l1, W_l2, b_l2):
  B = self_rows.shape[0]
  BLK = 256
  grid = (B // BLK,)
  f32 = jnp.float32
  inv_m = 1.0 / MAX_NEIGHBORS

  def dot(a, b):
    return lax.dot_general(a, b, (((1,), (0,)), ((), ())),
                           preferred_element_type=f32)

  def body(self_r, sc_r, sd_r, sg_r, cc_r, cd_r, cg_r,
           Wc_r, bc_r, Wd_r, bd_r, Wg_r, bg_r,
           Wl1_r, bl1_r, Wl2_r, bl2_r, out_r):
    h = dot(self_r[...], Wd_r[...]) + bd_r[...]
    agg_c = (dot(sc_r[...], Wc_r[...]) + cc_r[...] * bc_r[...]) * inv_m
    agg_d = (dot(sd_r[...], Wd_r[...]) + cd_r[...] * bd_r[...]) * inv_m
    agg_g = (dot(sg_r[...], Wg_r[...]) + cg_r[...] * bg_r[...]) * inv_m
    for Wl_r, bl_r in ((Wl1_r, bl1_r), (Wl2_r, bl2_r)):
      Wl = Wl_r[...]
      pre = (dot(h, Wl[0:EMBED_D]) + dot(agg_c, Wl[EMBED_D:2 * EMBED_D])
             + dot(agg_d, Wl[2 * EMBED_D:3 * EMBED_D])
             + dot(agg_g, Wl[3 * EMBED_D:4 * EMBED_D]) + bl_r[...])
      h = jnp.maximum(pre, 0.0)
    out_r[...] = h

  def rows_spec(d):
    return pl.BlockSpec((BLK, d), lambda i: (i, 0))

  def full_spec(shape):
    return pl.BlockSpec(shape, lambda i: tuple(0 for _ in shape))

  Dc, Dd, Dg = sums_c.shape[1], sums_d.shape[1], sums_g.shape[1]
  b2 = lambda v: v.reshape(1, EMBED_D)
  c2 = lambda v: v.reshape(B, 1)
  return pl.pallas_call(
      body,
      grid=grid,
      in_specs=[
          rows_spec(Dd), rows_spec(Dc), rows_spec(Dd), rows_spec(Dg),
          rows_spec(1), rows_spec(1), rows_spec(1),
          full_spec((Dc, EMBED_D)), full_spec((1, EMBED_D)),
          full_spec((Dd, EMBED_D)), full_spec((1, EMBED_D)),
          full_spec((Dg, EMBED_D)), full_spec((1, EMBED_D)),
          full_spec((4 * EMBED_D, EMBED_D)), full_spec((1, EMBED_D)),
          full_spec((4 * EMBED_D, EMBED_D)), full_spec((1, EMBED_D)),
      ],
      out_specs=rows_spec(EMBED_D),
      out_shape=jax.ShapeDtypeStruct((B, EMBED_D), f32),
  )(self_rows, sums_c, sums_d, sums_g, c2(cnt_c), c2(cnt_d), c2(cnt_g),
    W_cell, b2(b_cell), W_drug, b2(b_drug), W_gene, b2(b_gene),
    W_l1, b2(b_l1), W_l2, b2(b_l2))


def kernel(id_batch, neigh_cell, neigh_drug, neigh_gene,
           cell_features, drug_features, gene_features,
           W_cell, b_cell, W_drug, b_drug, W_gene, b_gene,
           W_l1, b_l1, W_l2, b_l2):
  def prep(neigh):
    mask = neigh != PAD_VALUE
    idx = jnp.where(mask, neigh, 0).astype(jnp.int32).reshape(-1)
    w = jnp.pad(mask.astype(jnp.float32),
                ((0, 0), (0, 16 - MAX_NEIGHBORS))).reshape(-1)
    cnt = mask.sum(axis=1).astype(jnp.float32)
    return idx, w, cnt

  idx_c, w_c, cnt_c = prep(neigh_cell)
  idx_d, w_d, cnt_d = prep(neigh_drug)
  idx_g, w_g, cnt_g = prep(neigh_gene)

  sums_c, sums_d, sums_g, self_rows = _sc_gather_sums(
      idx_c, w_c, idx_d, w_d, idx_g, w_g, id_batch.astype(jnp.int32),
      cell_features, drug_features, gene_features)

  return _tc_mlp(self_rows, sums_c, sums_d, sums_g, cnt_c, cnt_d, cnt_g,
                 W_cell, b_cell, W_drug, b_drug, W_gene, b_gene,
                 W_l1, b_l1, W_l2, b_l2)
